# Initial kernel scaffold; baseline (speedup 1.0000x reference)
import jax
import jax.numpy as jnp
from jax.experimental import pallas as pl


def kernel(x, edge_index, W_init, W0, b0, W1, b1):
    raise NotImplementedError("write your pallas kernel here")


# speedup vs baseline: 4.2312x; 4.2312x over previous
import functools

import jax
import jax.numpy as jnp
from jax import lax
from jax.experimental import pallas as pl
from jax.experimental.pallas import tpu as pltpu
from jax.experimental.pallas import tpu_sc as plsc

_N = 10000
_E = 320000
_D = 128
_BATCH = 10

_NC = 2
_NS = 16
_NW = _NC * _NS
_K = 128
_CHUNKS = _E // _K
_ITERS = -(-_CHUNKS // _NW)
_NP = 10240
_RPS = _NP // _NS
_L = 16

_mesh = plsc.VectorSubcoreMesh(core_axis_name="c", subcore_axis_name="s")


@functools.partial(
    pl.kernel,
    mesh=_mesh,
    out_type=jax.ShapeDtypeStruct((_NW, 1, 2 * _NP), jnp.float32),
    scratch_types=[
        pltpu.VMEM((_K,), jnp.int32),
        pltpu.VMEM((_K,), jnp.int32),
        pltpu.VMEM((2 * _NP,), jnp.float32),
    ],
    compiler_params=pltpu.CompilerParams(needs_layout_passes=False),
)
def _sc_degrees(src_hbm, dst_hbm, zeros_hbm, out_hbm, sidx, didx, hist):
    cid = lax.axis_index("c")
    sid = lax.axis_index("s")
    wid = sid * _NC + cid

    pltpu.sync_copy(zeros_hbm, hist)
    ones_v = jnp.ones((_L,), jnp.float32)

    def body(i, _):
        chunk = wid + i * _NW

        @pl.when(chunk < _CHUNKS)
        def _():
            base = chunk * _K
            pltpu.sync_copy(src_hbm.at[pl.ds(base, _K)], sidx)
            pltpu.sync_copy(dst_hbm.at[pl.ds(base, _K)], didx)
            for j in range(_K // _L):
                sv = sidx[pl.ds(j * _L, _L)]
                dv = didx[pl.ds(j * _L, _L)]
                plsc.addupdate_scatter(hist, [sv], ones_v)
                plsc.addupdate_scatter(hist, [dv + _NP], ones_v)

        return 0

    lax.fori_loop(0, _ITERS, body, 0)
    pltpu.sync_copy(hist, out_hbm.at[wid, 0])


@functools.partial(
    pl.kernel,
    mesh=_mesh,
    out_type=jax.ShapeDtypeStruct((_NC, _NP, _D), jnp.float32),
    scratch_types=[
        pltpu.VMEM((_K,), jnp.int32),
        pltpu.VMEM((_K,), jnp.int32),
        pltpu.VMEM((_K, _D), jnp.float32),
        pltpu.VMEM((_K, _D), jnp.float32),
        pltpu.VMEM_SHARED((_NP, _D), jnp.float32),
        pltpu.SemaphoreType.DMA,
    ],
)
def _sc_aggregate(m_hbm, src_hbm, dst_hbm, zeros_hbm, out_hbm, sidx, didx,
                  rows, zbuf, acc, sem):
    cid = lax.axis_index("c")
    sid = lax.axis_index("s")
    wid = sid * _NC + cid

    pltpu.sync_copy(zeros_hbm, zbuf)
    base_row = sid * _RPS
    for j in range(_RPS // _K):
        pltpu.sync_copy(zbuf, acc.at[pl.ds(base_row + j * _K, _K)])
    plsc.subcore_barrier()

    def body(i, _):
        chunk = wid + i * _NW

        @pl.when(chunk < _CHUNKS)
        def _():
            base = chunk * _K
            pltpu.sync_copy(src_hbm.at[pl.ds(base, _K)], sidx)
            pltpu.sync_copy(dst_hbm.at[pl.ds(base, _K)], didx)
            pltpu.async_copy(m_hbm.at[sidx], rows, sem).wait()
            pltpu.sync_copy(rows, acc.at[didx], add=True)

        return 0

    lax.fori_loop(0, _ITERS, body, 0)
    plsc.subcore_barrier()

    pltpu.sync_copy(acc.at[pl.ds(base_row, _RPS)],
                    out_hbm.at[cid, pl.ds(base_row, _RPS)])


_R = 400
_GRID = _N // _R


def _tc_embed_body(x_ref, w_ref, ds_ref, dd_ref, m_ref, ns_ref, nd_ref):
    ns = lax.rsqrt(jnp.maximum(jnp.sum(ds_ref[...], axis=0), 1.0))
    nd = lax.rsqrt(jnp.maximum(jnp.sum(dd_ref[...], axis=0), 1.0))
    ns_ref[...] = ns
    nd_ref[...] = nd
    h = jnp.dot(x_ref[...], w_ref[...], preferred_element_type=jnp.float32)
    m_ref[...] = h * ns


def _tc_embed(x, w_init, deg_s, deg_d):
    return pl.pallas_call(
        _tc_embed_body,
        grid=(_GRID,),
        in_specs=[
            pl.BlockSpec((_R, _D), lambda i: (i, 0)),
            pl.BlockSpec((_D, _D), lambda i: (0, 0)),
            pl.BlockSpec((_NW, _R, 1), lambda i: (0, i, 0)),
            pl.BlockSpec((_NW, _R, 1), lambda i: (0, i, 0)),
        ],
        out_specs=[
            pl.BlockSpec((_R, _D), lambda i: (i, 0)),
            pl.BlockSpec((_R, 1), lambda i: (i, 0)),
            pl.BlockSpec((_R, 1), lambda i: (i, 0)),
        ],
        out_shape=[
            jax.ShapeDtypeStruct((_N, _D), jnp.float32),
            jax.ShapeDtypeStruct((_N, 1), jnp.float32),
            jax.ShapeDtypeStruct((_N, 1), jnp.float32),
        ],
    )(x, w_init, deg_s, deg_d)


def _tc_layer_body(final, agg_ref, ns_ref, nd_ref, w_ref, b_ref, out_ref):
    agg = (agg_ref[0] + agg_ref[1]) * nd_ref[...]
    z = jnp.dot(agg, w_ref[...], preferred_element_type=jnp.float32)
    h = jnp.maximum(z + b_ref[...], 0.0)
    if not final:
        h = h * ns_ref[...]
    out_ref[...] = h


def _tc_layer(agg_parts, ns, nd, w, b, final):
    return pl.pallas_call(
        functools.partial(_tc_layer_body, final),
        grid=(_GRID,),
        in_specs=[
            pl.BlockSpec((_NC, _R, _D), lambda i: (0, i, 0)),
            pl.BlockSpec((_R, 1), lambda i: (i, 0)),
            pl.BlockSpec((_R, 1), lambda i: (i, 0)),
            pl.BlockSpec((_D, _D), lambda i: (0, 0)),
            pl.BlockSpec((1, _D), lambda i: (0, 0)),
        ],
        out_specs=pl.BlockSpec((_R, _D), lambda i: (i, 0)),
        out_shape=jax.ShapeDtypeStruct((_N, _D), jnp.float32),
    )(agg_parts, ns, nd, w, b.reshape(1, _D))


def kernel(x, edge_index, W_init, W0, b0, W1, b1):
    src = edge_index[0].astype(jnp.int32)
    dst = edge_index[1].astype(jnp.int32)
    zeros_h = jnp.zeros((2 * _NP,), jnp.float32)
    zeros_d = jnp.zeros((_K, _D), jnp.float32)
    deg_flat = _sc_degrees(src, dst, zeros_h)
    deg = deg_flat.reshape(_NW, 2, _NP, 1)
    m0, ns, nd = _tc_embed(x, W_init, deg[:, 0], deg[:, 1])
    agg1 = _sc_aggregate(m0, src, dst, zeros_d)
    m1 = _tc_layer(agg1, ns, nd, W0, b0, final=False)
    agg2 = _sc_aggregate(m1, src, dst, zeros_d)
    h2 = _tc_layer(agg2, ns, nd, W1, b1, final=True)
    return h2.reshape(_BATCH, -1, _D)


# speedup vs baseline: 6.3233x; 1.4944x over previous
import functools

import jax
import jax.numpy as jnp
from jax import lax
from jax.experimental import pallas as pl
from jax.experimental.pallas import tpu as pltpu
from jax.experimental.pallas import tpu_sc as plsc

_N = 10000
_E = 320000
_D = 128
_BATCH = 10

_NC = 2
_NS = 16
_NW = _NC * _NS
_K = 128
_DCH = _E // _K
_DCPW = 80
_KE = 64
_ACH = _E // _KE
_ACPW = 160
_AHALF = _ACPW // 2
_NP = 10240
_RPS = _NP // _NS
_L = 16

_mesh = plsc.VectorSubcoreMesh(core_axis_name="c", subcore_axis_name="s")


@functools.partial(
    pl.kernel,
    mesh=_mesh,
    out_type=jax.ShapeDtypeStruct((_NW, 1, 2 * _NP), jnp.float32),
    scratch_types=[
        pltpu.VMEM((_DCPW, 1, _K), jnp.int32),
        pltpu.VMEM((_DCPW, 1, _K), jnp.int32),
        pltpu.VMEM((2 * _NP,), jnp.float32),
    ],
    compiler_params=pltpu.CompilerParams(needs_layout_passes=False),
)
def _sc_degrees(src_hbm, dst_hbm, zeros_hbm, out_hbm, sidx, didx, hist):
    cid = lax.axis_index("c")
    sid = lax.axis_index("s")
    wid = sid * _NC + cid
    base = wid * _DCPW

    pltpu.sync_copy(zeros_hbm, hist)
    pltpu.sync_copy(src_hbm.at[pl.ds(base, _DCPW)], sidx)
    pltpu.sync_copy(dst_hbm.at[pl.ds(base, _DCPW)], didx)
    ones_v = jnp.ones((_L,), jnp.float32)

    def body(i, _):
        @pl.when(base + i < _DCH)
        def _():
            for j in range(_K // _L):
                sv = sidx[i, 0, pl.ds(j * _L, _L)]
                dv = didx[i, 0, pl.ds(j * _L, _L)]
                plsc.addupdate_scatter(hist, [sv], ones_v)
                plsc.addupdate_scatter(hist, [dv + _NP], ones_v)

        return 0

    lax.fori_loop(0, _DCPW, body, 0)
    pltpu.sync_copy(hist, out_hbm.at[wid, 0])


@functools.partial(
    pl.kernel,
    mesh=_mesh,
    out_type=jax.ShapeDtypeStruct((_NC, _NP, _D), jnp.float32),
    scratch_types=[
        pltpu.VMEM((_AHALF + 2, 1, _KE), jnp.int32),
        pltpu.VMEM((_AHALF, 1, _KE), jnp.int32),
        pltpu.VMEM((_KE, _D), jnp.float32),
        pltpu.VMEM((_KE, _D), jnp.float32),
        pltpu.VMEM_SHARED((_NP, _D), jnp.float32),
        pltpu.SemaphoreType.DMA,
        pltpu.SemaphoreType.DMA,
    ],
)
def _sc_aggregate(m_hbm, src_hbm, dst_hbm, zeros_hbm, out_hbm, sidx, didx,
                  rows0, rows1, acc, sem0, sem1):
    cid = lax.axis_index("c")
    sid = lax.axis_index("s")
    wid = sid * _NC + cid
    base = wid * _ACPW

    pltpu.sync_copy(zeros_hbm, rows0)
    base_row = sid * _RPS
    for j in range(_RPS // _KE):
        pltpu.sync_copy(rows0, acc.at[pl.ds(base_row + j * _KE, _KE)])
    plsc.subcore_barrier()

    for half in range(2):
        hbase = base + half * _AHALF
        pltpu.sync_copy(src_hbm.at[pl.ds(hbase, _AHALF)],
                        sidx.at[pl.ds(0, _AHALF)])
        pltpu.sync_copy(dst_hbm.at[pl.ds(hbase, _AHALF)], didx)

        @pl.when(hbase < _ACH)
        def _():
            pltpu.async_copy(m_hbm.at[sidx.at[0, 0]], rows0, sem0)

        @pl.when(hbase + 1 < _ACH)
        def _():
            pltpu.async_copy(m_hbm.at[sidx.at[1, 0]], rows1, sem1)

        def pair(p, _, hbase=hbase):
            i0 = 2 * p

            @pl.when(hbase + i0 < _ACH)
            def _():
                pltpu.make_async_copy(m_hbm.at[sidx.at[i0, 0]], rows0,
                                      sem0).wait()
                pltpu.sync_copy(rows0, acc.at[didx.at[i0, 0]], add=True)

            @pl.when((hbase + i0 + 2 < _ACH) & (i0 + 2 < _AHALF))
            def _():
                pltpu.async_copy(m_hbm.at[sidx.at[i0 + 2, 0]], rows0, sem0)

            i1 = i0 + 1

            @pl.when(hbase + i1 < _ACH)
            def _():
                pltpu.make_async_copy(m_hbm.at[sidx.at[i1, 0]], rows1,
                                      sem1).wait()
                pltpu.sync_copy(rows1, acc.at[didx.at[i1, 0]], add=True)

            @pl.when((hbase + i1 + 2 < _ACH) & (i1 + 2 < _AHALF))
            def _():
                pltpu.async_copy(m_hbm.at[sidx.at[i1 + 2, 0]], rows1, sem1)

            return 0

        lax.fori_loop(0, _AHALF // 2, pair, 0)

    plsc.subcore_barrier()

    pltpu.sync_copy(acc.at[pl.ds(base_row, _RPS)],
                    out_hbm.at[cid, pl.ds(base_row, _RPS)])


_R = 400
_GRID = _N // _R


def _tc_embed_body(x_ref, w_ref, ds_ref, dd_ref, m_ref, ns_ref, nd_ref):
    ns = lax.rsqrt(jnp.maximum(jnp.sum(ds_ref[...], axis=0), 1.0))
    nd = lax.rsqrt(jnp.maximum(jnp.sum(dd_ref[...], axis=0), 1.0))
    ns_ref[...] = ns
    nd_ref[...] = nd
    h = jnp.dot(x_ref[...], w_ref[...], preferred_element_type=jnp.float32)
    m_ref[...] = h * ns


def _tc_embed(x, w_init, deg_s, deg_d):
    return pl.pallas_call(
        _tc_embed_body,
        grid=(_GRID,),
        in_specs=[
            pl.BlockSpec((_R, _D), lambda i: (i, 0)),
            pl.BlockSpec((_D, _D), lambda i: (0, 0)),
            pl.BlockSpec((_NW, _R, 1), lambda i: (0, i, 0)),
            pl.BlockSpec((_NW, _R, 1), lambda i: (0, i, 0)),
        ],
        out_specs=[
            pl.BlockSpec((_R, _D), lambda i: (i, 0)),
            pl.BlockSpec((_R, 1), lambda i: (i, 0)),
            pl.BlockSpec((_R, 1), lambda i: (i, 0)),
        ],
        out_shape=[
            jax.ShapeDtypeStruct((_N, _D), jnp.float32),
            jax.ShapeDtypeStruct((_N, 1), jnp.float32),
            jax.ShapeDtypeStruct((_N, 1), jnp.float32),
        ],
    )(x, w_init, deg_s, deg_d)


def _tc_layer_body(final, agg_ref, ns_ref, nd_ref, w_ref, b_ref, out_ref):
    agg = (agg_ref[0] + agg_ref[1]) * nd_ref[...]
    z = jnp.dot(agg, w_ref[...], preferred_element_type=jnp.float32)
    h = jnp.maximum(z + b_ref[...], 0.0)
    if not final:
        h = h * ns_ref[...]
    out_ref[...] = h


def _tc_layer(agg_parts, ns, nd, w, b, final):
    return pl.pallas_call(
        functools.partial(_tc_layer_body, final),
        grid=(_GRID,),
        in_specs=[
            pl.BlockSpec((_NC, _R, _D), lambda i: (0, i, 0)),
            pl.BlockSpec((_R, 1), lambda i: (i, 0)),
            pl.BlockSpec((_R, 1), lambda i: (i, 0)),
            pl.BlockSpec((_D, _D), lambda i: (0, 0)),
            pl.BlockSpec((1, _D), lambda i: (0, 0)),
        ],
        out_specs=pl.BlockSpec((_R, _D), lambda i: (i, 0)),
        out_shape=jax.ShapeDtypeStruct((_N, _D), jnp.float32),
    )(agg_parts, ns, nd, w, b.reshape(1, _D))


def kernel(x, edge_index, W_init, W0, b0, W1, b1):
    src = edge_index[0].astype(jnp.int32)
    dst = edge_index[1].astype(jnp.int32)
    padd = ((0, _DCPW * _NW - _DCH), (0, 0), (0, 0))
    srcd = jnp.pad(src.reshape(_DCH, 1, _K), padd)
    dstd = jnp.pad(dst.reshape(_DCH, 1, _K), padd)
    pada = ((0, _ACPW * _NW - _ACH), (0, 0), (0, 0))
    srca = jnp.pad(src.reshape(_ACH, 1, _KE), pada)
    dsta = jnp.pad(dst.reshape(_ACH, 1, _KE), pada)
    zeros_h = jnp.zeros((2 * _NP,), jnp.float32)
    zeros_r = jnp.zeros((_KE, _D), jnp.float32)
    deg_flat = _sc_degrees(srcd, dstd, zeros_h)
    deg = deg_flat.reshape(_NW, 2, _NP, 1)
    m0, ns, nd = _tc_embed(x, W_init, deg[:, 0], deg[:, 1])
    agg1 = _sc_aggregate(m0, srca, dsta, zeros_r)
    m1 = _tc_layer(agg1, ns, nd, W0, b0, final=False)
    agg2 = _sc_aggregate(m1, srca, dsta, zeros_r)
    h2 = _tc_layer(agg2, ns, nd, W1, b1, final=True)
    return h2.reshape(_BATCH, -1, _D)


# speedup vs baseline: 9.1648x; 1.4494x over previous
import functools

import jax
import jax.numpy as jnp
from jax import lax
from jax.experimental import pallas as pl
from jax.experimental.pallas import tpu as pltpu
from jax.experimental.pallas import tpu_sc as plsc

_N = 10000
_E = 320000
_D = 128
_BATCH = 10

_NC = 2
_NS = 16
_NW = _NC * _NS
_K = 128
_DCH = _E // _K
_DCPW = 80
_KE = 64
_ACH = _E // _KE
_ACPW = 160
_AHALF = _ACPW // 2
_NP = 10240
_RPS = _NP // _NS
_L = 16
_CB = 2048

_mesh = plsc.VectorSubcoreMesh(core_axis_name="c", subcore_axis_name="s")


@functools.partial(
    pl.kernel,
    mesh=_mesh,
    out_type=jax.ShapeDtypeStruct((_NC, 2, _NP, 16), jnp.float32),
    scratch_types=[
        pltpu.VMEM((_DCPW, 1, _K), jnp.int32),
        pltpu.VMEM((_DCPW, 1, _K), jnp.int32),
        pltpu.VMEM((2 * _NP // _CB, 1, _CB), jnp.float32),
        pltpu.VMEM((16, 1, 128), jnp.float32),
        pltpu.VMEM((128, 16), jnp.float32),
        pltpu.VMEM_SHARED((_NS, 1, _CB), jnp.float32),
    ],
    compiler_params=pltpu.CompilerParams(needs_layout_passes=False),
)
def _sc_degrees(src_hbm, dst_hbm, zeros_hbm, out_hbm, sidx, didx, hist,
                redbuf, rowbuf, stage):
    cid = lax.axis_index("c")
    sid = lax.axis_index("s")
    wid = sid * _NC + cid
    base = wid * _DCPW

    pltpu.sync_copy(zeros_hbm, hist)
    pltpu.sync_copy(src_hbm.at[pl.ds(base, _DCPW)], sidx)
    pltpu.sync_copy(dst_hbm.at[pl.ds(base, _DCPW)], didx)
    ones_v = jnp.ones((_L,), jnp.float32)
    zero_i = jnp.zeros((_L,), jnp.int32)

    def body(i, _):
        @pl.when(base + i < _DCH)
        def _():
            for j in range(_K // _L):
                sv = sidx[i, 0, pl.ds(j * _L, _L)]
                dv = didx[i, 0, pl.ds(j * _L, _L)] + _NP
                plsc.addupdate_scatter(
                    hist, [sv >> 11, zero_i, sv & (_CB - 1)], ones_v)
                plsc.addupdate_scatter(
                    hist, [dv >> 11, zero_i, dv & (_CB - 1)], ones_v)

        return 0

    lax.fori_loop(0, _DCPW, body, 0)

    for r in range(2 * _NP // _CB):
        pltpu.sync_copy(hist.at[r], stage.at[sid])
        plsc.subcore_barrier()
        pltpu.sync_copy(stage.at[:, :, pl.ds(sid * 128, 128)], redbuf)
        zcol = jnp.zeros((_L,), jnp.int32)
        for c in range(128 // _L):
            acc = redbuf[0, 0, pl.ds(c * _L, _L)]
            for t in range(1, _NS):
                acc = acc + redbuf[t, 0, pl.ds(c * _L, _L)]
            rows_i = lax.iota(jnp.int32, _L) + c * _L
            plsc.store_scatter(rowbuf, [rows_i, zcol], acc)
        h = 0 if r * _CB < _NP else 1
        node = r * _CB + sid * 128 - h * _NP
        pltpu.sync_copy(rowbuf, out_hbm.at[cid, h, pl.ds(node, 128)])
        plsc.subcore_barrier()


@functools.partial(
    pl.kernel,
    mesh=_mesh,
    out_type=jax.ShapeDtypeStruct((_NC, _NP, _D), jnp.float32),
    scratch_types=[
        pltpu.VMEM((_AHALF + 2, 1, _KE), jnp.int32),
        pltpu.VMEM((_AHALF, 1, _KE), jnp.int32),
        pltpu.VMEM((_KE, _D), jnp.float32),
        pltpu.VMEM((_KE, _D), jnp.float32),
        pltpu.VMEM_SHARED((_NP, _D), jnp.float32),
        pltpu.SemaphoreType.DMA,
        pltpu.SemaphoreType.DMA,
    ],
)
def _sc_aggregate(m_hbm, src_hbm, dst_hbm, zeros_hbm, out_hbm, sidx, didx,
                  rows0, rows1, acc, sem0, sem1):
    cid = lax.axis_index("c")
    sid = lax.axis_index("s")
    wid = sid * _NC + cid
    base = wid * _ACPW

    pltpu.sync_copy(zeros_hbm, rows0)
    base_row = sid * _RPS
    for j in range(_RPS // _KE):
        pltpu.sync_copy(rows0, acc.at[pl.ds(base_row + j * _KE, _KE)])
    plsc.subcore_barrier()

    for half in range(2):
        hbase = base + half * _AHALF
        pltpu.sync_copy(src_hbm.at[pl.ds(hbase, _AHALF)],
                        sidx.at[pl.ds(0, _AHALF)])
        pltpu.sync_copy(dst_hbm.at[pl.ds(hbase, _AHALF)], didx)

        @pl.when(hbase < _ACH)
        def _():
            pltpu.async_copy(m_hbm.at[sidx.at[0, 0]], rows0, sem0)

        @pl.when(hbase + 1 < _ACH)
        def _():
            pltpu.async_copy(m_hbm.at[sidx.at[1, 0]], rows1, sem1)

        def pair(p, _, hbase=hbase):
            i0 = 2 * p

            @pl.when(hbase + i0 < _ACH)
            def _():
                pltpu.make_async_copy(m_hbm.at[sidx.at[i0, 0]], rows0,
                                      sem0).wait()
                pltpu.sync_copy(rows0, acc.at[didx.at[i0, 0]], add=True)

            @pl.when((hbase + i0 + 2 < _ACH) & (i0 + 2 < _AHALF))
            def _():
                pltpu.async_copy(m_hbm.at[sidx.at[i0 + 2, 0]], rows0, sem0)

            i1 = i0 + 1

            @pl.when(hbase + i1 < _ACH)
            def _():
                pltpu.make_async_copy(m_hbm.at[sidx.at[i1, 0]], rows1,
                                      sem1).wait()
                pltpu.sync_copy(rows1, acc.at[didx.at[i1, 0]], add=True)

            @pl.when((hbase + i1 + 2 < _ACH) & (i1 + 2 < _AHALF))
            def _():
                pltpu.async_copy(m_hbm.at[sidx.at[i1 + 2, 0]], rows1, sem1)

            return 0

        lax.fori_loop(0, _AHALF // 2, pair, 0)

    plsc.subcore_barrier()

    pltpu.sync_copy(acc.at[pl.ds(base_row, _RPS)],
                    out_hbm.at[cid, pl.ds(base_row, _RPS)])


_R = 400
_GRID = _N // _R


def _tc_embed_body(x_ref, w_ref, deg_ref, m_ref, ns_ref, nd_ref):
    ns = lax.rsqrt(jnp.maximum(
        deg_ref[0, 0, :, :1] + deg_ref[1, 0, :, :1], 1.0))
    nd = lax.rsqrt(jnp.maximum(
        deg_ref[0, 1, :, :1] + deg_ref[1, 1, :, :1], 1.0))
    ns_ref[...] = ns
    nd_ref[...] = nd
    h = jnp.dot(x_ref[...], w_ref[...], preferred_element_type=jnp.float32)
    m_ref[...] = h * ns


def _tc_embed(x, w_init, deg_parts):
    return pl.pallas_call(
        _tc_embed_body,
        grid=(_GRID,),
        in_specs=[
            pl.BlockSpec((_R, _D), lambda i: (i, 0)),
            pl.BlockSpec((_D, _D), lambda i: (0, 0)),
            pl.BlockSpec((_NC, 2, _R, 16), lambda i: (0, 0, i, 0)),
        ],
        out_specs=[
            pl.BlockSpec((_R, _D), lambda i: (i, 0)),
            pl.BlockSpec((_R, 1), lambda i: (i, 0)),
            pl.BlockSpec((_R, 1), lambda i: (i, 0)),
        ],
        out_shape=[
            jax.ShapeDtypeStruct((_N, _D), jnp.float32),
            jax.ShapeDtypeStruct((_N, 1), jnp.float32),
            jax.ShapeDtypeStruct((_N, 1), jnp.float32),
        ],
    )(x, w_init, deg_parts)


def _tc_layer_body(final, agg_ref, ns_ref, nd_ref, w_ref, b_ref, out_ref):
    agg = (agg_ref[0] + agg_ref[1]) * nd_ref[...]
    z = jnp.dot(agg, w_ref[...], preferred_element_type=jnp.float32)
    h = jnp.maximum(z + b_ref[...], 0.0)
    if not final:
        h = h * ns_ref[...]
    out_ref[...] = h


def _tc_layer(agg_parts, ns, nd, w, b, final):
    return pl.pallas_call(
        functools.partial(_tc_layer_body, final),
        grid=(_GRID,),
        in_specs=[
            pl.BlockSpec((_NC, _R, _D), lambda i: (0, i, 0)),
            pl.BlockSpec((_R, 1), lambda i: (i, 0)),
            pl.BlockSpec((_R, 1), lambda i: (i, 0)),
            pl.BlockSpec((_D, _D), lambda i: (0, 0)),
            pl.BlockSpec((1, _D), lambda i: (0, 0)),
        ],
        out_specs=pl.BlockSpec((_R, _D), lambda i: (i, 0)),
        out_shape=jax.ShapeDtypeStruct((_N, _D), jnp.float32),
    )(agg_parts, ns, nd, w, b.reshape(1, _D))


def kernel(x, edge_index, W_init, W0, b0, W1, b1):
    src = edge_index[0].astype(jnp.int32)
    dst = edge_index[1].astype(jnp.int32)
    padd = ((0, _DCPW * _NW - _DCH), (0, 0), (0, 0))
    srcd = jnp.pad(src.reshape(_DCH, 1, _K), padd)
    dstd = jnp.pad(dst.reshape(_DCH, 1, _K), padd)
    pada = ((0, _ACPW * _NW - _ACH), (0, 0), (0, 0))
    srca = jnp.pad(src.reshape(_ACH, 1, _KE), pada)
    dsta = jnp.pad(dst.reshape(_ACH, 1, _KE), pada)
    zeros_h = jnp.zeros((2 * _NP // _CB, 1, _CB), jnp.float32)
    zeros_r = jnp.zeros((_KE, _D), jnp.float32)
    deg_parts = _sc_degrees(srcd, dstd, zeros_h)
    m0, ns, nd = _tc_embed(x, W_init, deg_parts)
    agg1 = _sc_aggregate(m0, srca, dsta, zeros_r)
    m1 = _tc_layer(agg1, ns, nd, W0, b0, final=False)
    agg2 = _sc_aggregate(m1, srca, dsta, zeros_r)
    h2 = _tc_layer(agg2, ns, nd, W1, b1, final=True)
    return h2.reshape(_BATCH, -1, _D)


# speedup vs baseline: 9.1681x; 1.0004x over previous
import functools

import jax
import jax.numpy as jnp
from jax import lax
from jax.experimental import pallas as pl
from jax.experimental.pallas import tpu as pltpu
from jax.experimental.pallas import tpu_sc as plsc

_N = 10000
_E = 320000
_D = 128
_BATCH = 10

_NC = 2
_NS = 16
_NW = _NC * _NS
_K = 128
_DCH = _E // _K
_DCPW = 80
_KE = 64
_ACH = _E // _KE
_ACPW = 160
_AHALF = _ACPW // 2
_NP = 10240
_RPS = _NP // _NS
_L = 16
_CB = 2048

_mesh = plsc.VectorSubcoreMesh(core_axis_name="c", subcore_axis_name="s")


@functools.partial(
    pl.kernel,
    mesh=_mesh,
    out_type=jax.ShapeDtypeStruct((_NC, 2, _NP, 16), jnp.float32),
    scratch_types=[
        pltpu.VMEM((_DCPW, 1, _K), jnp.int32),
        pltpu.VMEM((_DCPW, 1, _K), jnp.int32),
        pltpu.VMEM((2 * _NP // _CB, 1, _CB), jnp.float32),
        pltpu.VMEM((16, 1, 128), jnp.float32),
        pltpu.VMEM((128, 16), jnp.float32),
        pltpu.VMEM_SHARED((_NS, 1, _CB), jnp.float32),
    ],
    compiler_params=pltpu.CompilerParams(needs_layout_passes=False),
)
def _sc_degrees(src_hbm, dst_hbm, zeros_hbm, out_hbm, sidx, didx, hist,
                redbuf, rowbuf, stage):
    cid = lax.axis_index("c")
    sid = lax.axis_index("s")
    wid = sid * _NC + cid
    base = wid * _DCPW

    pltpu.sync_copy(zeros_hbm, hist)
    pltpu.sync_copy(src_hbm.at[pl.ds(base, _DCPW)], sidx)
    pltpu.sync_copy(dst_hbm.at[pl.ds(base, _DCPW)], didx)
    ones_v = jnp.ones((_L,), jnp.float32)
    zero_i = jnp.zeros((_L,), jnp.int32)

    def body(i, _):
        @pl.when(base + i < _DCH)
        def _():
            for j in range(_K // _L):
                sv = sidx[i, 0, pl.ds(j * _L, _L)]
                dv = didx[i, 0, pl.ds(j * _L, _L)] + _NP
                plsc.addupdate_scatter(
                    hist, [sv >> 11, zero_i, sv & (_CB - 1)], ones_v)
                plsc.addupdate_scatter(
                    hist, [dv >> 11, zero_i, dv & (_CB - 1)], ones_v)

        return 0

    lax.fori_loop(0, _DCPW, body, 0)

    for r in range(2 * _NP // _CB):
        pltpu.sync_copy(hist.at[r], stage.at[sid])
        plsc.subcore_barrier()
        pltpu.sync_copy(stage.at[:, :, pl.ds(sid * 128, 128)], redbuf)
        zcol = jnp.zeros((_L,), jnp.int32)
        for c in range(128 // _L):
            acc = redbuf[0, 0, pl.ds(c * _L, _L)]
            for t in range(1, _NS):
                acc = acc + redbuf[t, 0, pl.ds(c * _L, _L)]
            rows_i = lax.iota(jnp.int32, _L) + c * _L
            plsc.store_scatter(rowbuf, [rows_i, zcol], acc)
        h = 0 if r * _CB < _NP else 1
        node = r * _CB + sid * 128 - h * _NP
        pltpu.sync_copy(rowbuf, out_hbm.at[cid, h, pl.ds(node, 128)])
        plsc.subcore_barrier()


@functools.partial(
    pl.kernel,
    mesh=_mesh,
    out_type=jax.ShapeDtypeStruct((_NC, _NP, _D), jnp.float32),
    scratch_types=[
        pltpu.VMEM((_AHALF + 2, 1, _KE), jnp.int32),
        pltpu.VMEM((_AHALF, 1, _KE), jnp.int32),
        pltpu.VMEM((_KE, _D), jnp.float32),
        pltpu.VMEM((_KE, _D), jnp.float32),
        pltpu.VMEM_SHARED((_NP, _D), jnp.float32),
        pltpu.SemaphoreType.DMA,
        pltpu.SemaphoreType.DMA,
    ],
)
def _sc_aggregate(m_hbm, src_hbm, dst_hbm, zeros_hbm, out_hbm, sidx, didx,
                  rows0, rows1, acc, sem0, sem1):
    cid = lax.axis_index("c")
    sid = lax.axis_index("s")
    wid = sid * _NC + cid
    base = wid * _ACPW

    pltpu.sync_copy(zeros_hbm, rows0)
    base_row = sid * _RPS
    for j in range(_RPS // _KE):
        pltpu.sync_copy(rows0, acc.at[pl.ds(base_row + j * _KE, _KE)])
    plsc.subcore_barrier()

    for half in range(2):
        hbase = base + half * _AHALF
        pltpu.sync_copy(src_hbm.at[pl.ds(hbase, _AHALF)],
                        sidx.at[pl.ds(0, _AHALF)])
        pltpu.sync_copy(dst_hbm.at[pl.ds(hbase, _AHALF)], didx)

        @pl.when(hbase < _ACH)
        def _():
            pltpu.async_copy(m_hbm.at[sidx.at[0, 0]], rows0, sem0)

        @pl.when(hbase + 1 < _ACH)
        def _():
            pltpu.async_copy(m_hbm.at[sidx.at[1, 0]], rows1, sem1)

        def pair(p, _, hbase=hbase):
            i0 = 2 * p

            @pl.when(hbase + i0 < _ACH)
            def _():
                pltpu.make_async_copy(m_hbm.at[sidx.at[i0, 0]], rows0,
                                      sem0).wait()
                pltpu.sync_copy(rows0, acc.at[didx.at[i0, 0]], add=True)

            @pl.when((hbase + i0 + 2 < _ACH) & (i0 + 2 < _AHALF))
            def _():
                pltpu.async_copy(m_hbm.at[sidx.at[i0 + 2, 0]], rows0, sem0)

            i1 = i0 + 1

            @pl.when(hbase + i1 < _ACH)
            def _():
                pltpu.make_async_copy(m_hbm.at[sidx.at[i1, 0]], rows1,
                                      sem1).wait()
                pltpu.sync_copy(rows1, acc.at[didx.at[i1, 0]], add=True)

            @pl.when((hbase + i1 + 2 < _ACH) & (i1 + 2 < _AHALF))
            def _():
                pltpu.async_copy(m_hbm.at[sidx.at[i1 + 2, 0]], rows1, sem1)

            return 0

        lax.fori_loop(0, _AHALF // 2, pair, 0)

    plsc.subcore_barrier()

    pltpu.sync_copy(acc.at[pl.ds(base_row, _RPS)],
                    out_hbm.at[cid, pl.ds(base_row, _RPS)])


_R = 400
_GRID = _N // _R


def _tc_h0_body(x_ref, w_ref, h_ref):
    h_ref[...] = jnp.dot(x_ref[...], w_ref[...],
                         preferred_element_type=jnp.float32)


def _tc_h0(x, w_init):
    return pl.pallas_call(
        _tc_h0_body,
        grid=(_GRID,),
        in_specs=[
            pl.BlockSpec((_R, _D), lambda i: (i, 0)),
            pl.BlockSpec((_D, _D), lambda i: (0, 0)),
        ],
        out_specs=pl.BlockSpec((_R, _D), lambda i: (i, 0)),
        out_shape=jax.ShapeDtypeStruct((_N, _D), jnp.float32),
    )(x, w_init)


def _tc_scale_body(h_ref, deg_ref, m_ref, ns_ref, nd_ref):
    ns = lax.rsqrt(jnp.maximum(
        deg_ref[0, 0, :, :1] + deg_ref[1, 0, :, :1], 1.0))
    nd = lax.rsqrt(jnp.maximum(
        deg_ref[0, 1, :, :1] + deg_ref[1, 1, :, :1], 1.0))
    ns_ref[...] = ns
    nd_ref[...] = nd
    m_ref[...] = h_ref[...] * ns


def _tc_embed(h0, deg_parts):
    return pl.pallas_call(
        _tc_scale_body,
        grid=(_GRID,),
        in_specs=[
            pl.BlockSpec((_R, _D), lambda i: (i, 0)),
            pl.BlockSpec((_NC, 2, _R, 16), lambda i: (0, 0, i, 0)),
        ],
        out_specs=[
            pl.BlockSpec((_R, _D), lambda i: (i, 0)),
            pl.BlockSpec((_R, 1), lambda i: (i, 0)),
            pl.BlockSpec((_R, 1), lambda i: (i, 0)),
        ],
        out_shape=[
            jax.ShapeDtypeStruct((_N, _D), jnp.float32),
            jax.ShapeDtypeStruct((_N, 1), jnp.float32),
            jax.ShapeDtypeStruct((_N, 1), jnp.float32),
        ],
    )(h0, deg_parts)


def _tc_layer_body(final, agg_ref, ns_ref, nd_ref, w_ref, b_ref, out_ref):
    agg = (agg_ref[0] + agg_ref[1]) * nd_ref[...]
    z = jnp.dot(agg, w_ref[...], preferred_element_type=jnp.float32)
    h = jnp.maximum(z + b_ref[...], 0.0)
    if not final:
        h = h * ns_ref[...]
    out_ref[...] = h


def _tc_layer(agg_parts, ns, nd, w, b, final):
    return pl.pallas_call(
        functools.partial(_tc_layer_body, final),
        grid=(_GRID,),
        in_specs=[
            pl.BlockSpec((_NC, _R, _D), lambda i: (0, i, 0)),
            pl.BlockSpec((_R, 1), lambda i: (i, 0)),
            pl.BlockSpec((_R, 1), lambda i: (i, 0)),
            pl.BlockSpec((_D, _D), lambda i: (0, 0)),
            pl.BlockSpec((1, _D), lambda i: (0, 0)),
        ],
        out_specs=pl.BlockSpec((_R, _D), lambda i: (i, 0)),
        out_shape=jax.ShapeDtypeStruct((_N, _D), jnp.float32),
    )(agg_parts, ns, nd, w, b.reshape(1, _D))


def kernel(x, edge_index, W_init, W0, b0, W1, b1):
    src = edge_index[0].astype(jnp.int32)
    dst = edge_index[1].astype(jnp.int32)
    padd = ((0, _DCPW * _NW - _DCH), (0, 0), (0, 0))
    srcd = jnp.pad(src.reshape(_DCH, 1, _K), padd)
    dstd = jnp.pad(dst.reshape(_DCH, 1, _K), padd)
    pada = ((0, _ACPW * _NW - _ACH), (0, 0), (0, 0))
    srca = jnp.pad(src.reshape(_ACH, 1, _KE), pada)
    dsta = jnp.pad(dst.reshape(_ACH, 1, _KE), pada)
    zeros_h = jnp.zeros((2 * _NP // _CB, 1, _CB), jnp.float32)
    zeros_r = jnp.zeros((_KE, _D), jnp.float32)
    deg_parts = _sc_degrees(srcd, dstd, zeros_h)
    h0 = _tc_h0(x, W_init)
    m0, ns, nd = _tc_embed(h0, deg_parts)
    agg1 = _sc_aggregate(m0, srca, dsta, zeros_r)
    m1 = _tc_layer(agg1, ns, nd, W0, b0, final=False)
    agg2 = _sc_aggregate(m1, srca, dsta, zeros_r)
    h2 = _tc_layer(agg2, ns, nd, W1, b1, final=True)
    return h2.reshape(_BATCH, -1, _D)
